# baseline (speedup 1.0000x reference)
import math

import jax
import jax.numpy as jnp
from jax.experimental import pallas as pl
from jax.experimental.pallas import tpu as pltpu

S, D, H, R = 2048, 1024, 16, 32
HR = H * R
BQ = 256
BK = 256
NBQ = S // BQ
NBK = S // BK
AW = 64
VW = H * AW
YW = HR + HR + VW
NEG = float(jnp.finfo(jnp.float32).min)
SCALE = 1.0 / math.sqrt(R)


def _qkv_kernel(x_ref, w_ref, b_ref, y_ref):
    y = jax.lax.dot_general(x_ref[...], w_ref[...], (((1,), (0,)), ((), ())),
                            preferred_element_type=jnp.float32)
    y_ref[...] = (y + b_ref[...]).astype(jnp.bfloat16)


def _attn_kernel(q_ref, k_ref, v_ref, wo_ref, bo_ref, out_ref, acc_ref):
    i = pl.program_id(0)
    j = pl.program_id(1)

    @pl.when(j == 0)
    def _init():
        acc_ref[...] = jnp.zeros_like(acc_ref)

    @pl.when(j <= i)
    def _compute():
        q = q_ref[...]
        k = k_ref[...]
        v = v_ref[...]
        row = i * BQ + jax.lax.broadcasted_iota(jnp.int32, (BQ, BK), 0)
        col = j * BK + jax.lax.broadcasted_iota(jnp.int32, (BQ, BK), 1)
        bias = jnp.where(row >= col, 0.0, NEG)
        for h in range(H):
            qh = q[:, h * R:(h + 1) * R]
            kh = k[:, h * R:(h + 1) * R]
            vh = v[:, h * AW:(h + 1) * AW]
            s = jax.lax.dot_general(qh, kh, (((1,), (1,)), ((), ())),
                                    preferred_element_type=jnp.float32)
            e = jnp.exp(s + bias).astype(jnp.bfloat16)
            oh = jax.lax.dot_general(e, vh, (((1,), (0,)), ((), ())),
                                     preferred_element_type=jnp.float32)
            acc_ref[:, h * AW:(h + 1) * AW] = acc_ref[:, h * AW:(h + 1) * AW] + oh

    @pl.when(j == i)
    def _finalize():
        outs = []
        for h in range(H):
            blk = acc_ref[:, h * AW:(h + 1) * AW]
            outs.append((blk[:, :R] / blk[:, R:R + 1]).astype(jnp.bfloat16))
        o = jnp.concatenate(outs, axis=1)
        out_ref[...] = jax.lax.dot_general(
            o, wo_ref[...], (((1,), (0,)), ((), ())),
            preferred_element_type=jnp.float32) + bo_ref[...]


def kernel(x, Wq, bq, Wk, bk, Wv, bv, Wo, bo):
    B = x.shape[0]
    x2 = x.reshape(S, D).astype(jnp.bfloat16)

    wv_aug = jnp.pad(Wv.reshape(D, H, R), ((0, 0), (0, 0), (0, AW - R)))
    bv_aug = jnp.concatenate(
        [bv.reshape(H, R), jnp.ones((H, 1), jnp.float32),
         jnp.zeros((H, AW - R - 1), jnp.float32)], axis=1)
    w_all = jnp.concatenate(
        [Wq * SCALE, Wk, wv_aug.reshape(D, VW)], axis=1).astype(jnp.bfloat16)
    b_all = jnp.concatenate(
        [bq * SCALE, bk, bv_aug.reshape(VW)]).reshape(1, YW)
    wo = Wo.astype(jnp.bfloat16)
    bo2 = bo.reshape(1, D)

    y = pl.pallas_call(
        _qkv_kernel,
        grid=(NBQ,),
        in_specs=[
            pl.BlockSpec((BQ, D), lambda i: (i, 0)),
            pl.BlockSpec((D, YW), lambda i: (0, 0)),
            pl.BlockSpec((1, YW), lambda i: (0, 0)),
        ],
        out_specs=pl.BlockSpec((BQ, YW), lambda i: (i, 0)),
        out_shape=jax.ShapeDtypeStruct((S, YW), jnp.bfloat16),
    )(x2, w_all, b_all)

    out = pl.pallas_call(
        _attn_kernel,
        grid=(NBQ, NBK),
        in_specs=[
            pl.BlockSpec((BQ, HR), lambda i, j: (i, 0)),
            pl.BlockSpec((BK, HR), lambda i, j: (j, 1)),
            pl.BlockSpec((BK, VW), lambda i, j: (j, 1)),
            pl.BlockSpec((HR, D), lambda i, j: (0, 0)),
            pl.BlockSpec((1, D), lambda i, j: (0, 0)),
        ],
        out_specs=pl.BlockSpec((BQ, D), lambda i, j: (i, 0)),
        out_shape=jax.ShapeDtypeStruct((S, D), jnp.float32),
        scratch_shapes=[pltpu.VMEM((BQ, H * AW), jnp.float32)],
    )(y, y, y, wo, bo2)

    return out.reshape(B, S, D)


# speedup vs baseline: 1.0199x; 1.0199x over previous
import math

import jax
import jax.numpy as jnp
from jax.experimental import pallas as pl
from jax.experimental.pallas import tpu as pltpu

S, D, H, R = 2048, 1024, 16, 32
HR = H * R
BQ = 256
BK = 256
NBQ = S // BQ
NBK = S // BK
AW = 64
VW = H * AW
YW = HR + HR + VW
NEG = float(jnp.finfo(jnp.float32).min)
SCALE = 1.0 / math.sqrt(R)


def _qkv_kernel(x_ref, wq_ref, wk_ref, wv_ref, bq_ref, bk_ref, bv_ref,
                q_ref, k_ref, v_ref):
    x = x_ref[...]
    q = jax.lax.dot_general(x, wq_ref[...], (((1,), (0,)), ((), ())),
                            preferred_element_type=jnp.float32)
    k = jax.lax.dot_general(x, wk_ref[...], (((1,), (0,)), ((), ())),
                            preferred_element_type=jnp.float32)
    v = jax.lax.dot_general(x, wv_ref[...], (((1,), (0,)), ((), ())),
                            preferred_element_type=jnp.float32)
    q_ref[...] = (q + bq_ref[...]).astype(jnp.bfloat16)
    k_ref[...] = (k + bk_ref[...]).astype(jnp.bfloat16)
    v_ref[...] = (v + bv_ref[...]).astype(jnp.bfloat16)


def _attn_kernel(q_ref, k_ref, v_ref, wo_ref, bo_ref, out_ref, acc_ref):
    i = pl.program_id(0)
    j = pl.program_id(1)

    @pl.when(j == 0)
    def _init():
        acc_ref[...] = jnp.zeros_like(acc_ref)

    @pl.when(j <= i)
    def _compute():
        q = q_ref[...]
        k = k_ref[...]
        v = v_ref[...]
        row = i * BQ + jax.lax.broadcasted_iota(jnp.int32, (BQ, BK), 0)
        col = j * BK + jax.lax.broadcasted_iota(jnp.int32, (BQ, BK), 1)
        bias = jnp.where(row >= col, 0.0, NEG)
        for h in range(H):
            qh = q[:, h * R:(h + 1) * R]
            kh = k[:, h * R:(h + 1) * R]
            vh = v[:, h * AW:(h + 1) * AW]
            s = jax.lax.dot_general(qh, kh, (((1,), (1,)), ((), ())),
                                    preferred_element_type=jnp.float32)
            e = jnp.exp(s + bias).astype(jnp.bfloat16)
            oh = jax.lax.dot_general(e, vh, (((1,), (0,)), ((), ())),
                                     preferred_element_type=jnp.float32)
            acc_ref[:, h * AW:(h + 1) * AW] = acc_ref[:, h * AW:(h + 1) * AW] + oh

    @pl.when(j == i)
    def _finalize():
        outs = []
        for h in range(H):
            blk = acc_ref[:, h * AW:(h + 1) * AW]
            outs.append((blk[:, :R] / blk[:, R:R + 1]).astype(jnp.bfloat16))
        o = jnp.concatenate(outs, axis=1)
        out_ref[...] = jax.lax.dot_general(
            o, wo_ref[...], (((1,), (0,)), ((), ())),
            preferred_element_type=jnp.float32) + bo_ref[...]


def kernel(x, Wq, bq, Wk, bk, Wv, bv, Wo, bo):
    B = x.shape[0]
    x2 = x.reshape(S, D).astype(jnp.bfloat16)

    wv_aug = jnp.pad(Wv.reshape(D, H, R), ((0, 0), (0, 0), (0, AW - R)))
    bv_aug = jnp.concatenate(
        [bv.reshape(H, R), jnp.ones((H, 1), jnp.float32),
         jnp.zeros((H, AW - R - 1), jnp.float32)], axis=1)
    wq2 = (Wq * SCALE).astype(jnp.bfloat16)
    wk2 = Wk.astype(jnp.bfloat16)
    wv2 = wv_aug.reshape(D, VW).astype(jnp.bfloat16)
    bq2 = (bq * SCALE).reshape(1, HR)
    bk2 = bk.reshape(1, HR)
    bv2 = bv_aug.reshape(1, VW)
    wo = Wo.astype(jnp.bfloat16)
    bo2 = bo.reshape(1, D)

    q, k, v = pl.pallas_call(
        _qkv_kernel,
        grid=(NBQ,),
        in_specs=[
            pl.BlockSpec((BQ, D), lambda i: (i, 0)),
            pl.BlockSpec((D, HR), lambda i: (0, 0)),
            pl.BlockSpec((D, HR), lambda i: (0, 0)),
            pl.BlockSpec((D, VW), lambda i: (0, 0)),
            pl.BlockSpec((1, HR), lambda i: (0, 0)),
            pl.BlockSpec((1, HR), lambda i: (0, 0)),
            pl.BlockSpec((1, VW), lambda i: (0, 0)),
        ],
        out_specs=[
            pl.BlockSpec((BQ, HR), lambda i: (i, 0)),
            pl.BlockSpec((BQ, HR), lambda i: (i, 0)),
            pl.BlockSpec((BQ, VW), lambda i: (i, 0)),
        ],
        out_shape=[jax.ShapeDtypeStruct((S, HR), jnp.bfloat16),
                   jax.ShapeDtypeStruct((S, HR), jnp.bfloat16),
                   jax.ShapeDtypeStruct((S, VW), jnp.bfloat16)],
    )(x2, wq2, wk2, wv2, bq2, bk2, bv2)

    out = pl.pallas_call(
        _attn_kernel,
        grid=(NBQ, NBK),
        in_specs=[
            pl.BlockSpec((BQ, HR), lambda i, j: (i, 0)),
            pl.BlockSpec((BK, HR), lambda i, j: (j, 0)),
            pl.BlockSpec((BK, VW), lambda i, j: (j, 0)),
            pl.BlockSpec((HR, D), lambda i, j: (0, 0)),
            pl.BlockSpec((1, D), lambda i, j: (0, 0)),
        ],
        out_specs=pl.BlockSpec((BQ, D), lambda i, j: (i, 0)),
        out_shape=jax.ShapeDtypeStruct((S, D), jnp.float32),
        scratch_shapes=[pltpu.VMEM((BQ, H * AW), jnp.float32)],
    )(q, k, v, wo, bo2)

    return out.reshape(B, S, D)


# speedup vs baseline: 1.4012x; 1.3739x over previous
import math

import jax
import jax.numpy as jnp
from jax.experimental import pallas as pl
from jax.experimental.pallas import tpu as pltpu

S, D, H, R = 2048, 1024, 16, 32
HR = H * R
BQ = 512
BK = 512
NBQ = S // BQ
NBK = S // BK
AW = 64
NEG = float(jnp.finfo(jnp.float32).min)
SCALE = 1.0 / math.sqrt(R)


def _qkv_kernel(x_ref, wq_ref, wk_ref, wv_ref, bq_ref, bk_ref, bv_ref,
                q_ref, k_ref, v_ref):
    x = x_ref[...]
    q = jax.lax.dot_general(x, wq_ref[...], (((1,), (0,)), ((), ())),
                            preferred_element_type=jnp.float32)
    k = jax.lax.dot_general(x, wk_ref[...], (((1,), (0,)), ((), ())),
                            preferred_element_type=jnp.float32)
    v = jax.lax.dot_general(x, wv_ref[...], (((1,), (0,)), ((), ())),
                            preferred_element_type=jnp.float32)
    q_ref[...] = (q + bq_ref[...]).astype(jnp.bfloat16)
    k_ref[...] = (k + bk_ref[...]).astype(jnp.bfloat16)
    v_ref[...] = (v + bv_ref[...]).astype(jnp.bfloat16)


def _attn_kernel(q_ref, k_ref, v_ref, wo_ref, bo_ref, out_ref, acc_ref):
    i = pl.program_id(0)
    j = pl.program_id(1)

    @pl.when(j == 0)
    def _init():
        acc_ref[...] = jnp.zeros_like(acc_ref)

    @pl.when(j <= i)
    def _compute():
        q = q_ref[...]
        k = k_ref[...]
        v = v_ref[...]
        row = i * BQ + jax.lax.broadcasted_iota(jnp.int32, (BQ, BK), 0)
        col = j * BK + jax.lax.broadcasted_iota(jnp.int32, (BQ, BK), 1)
        bias = jnp.where(row >= col, 0.0, NEG)
        aug = (jax.lax.broadcasted_iota(jnp.int32, (BK, AW - R), 1)
               == 0).astype(jnp.bfloat16)
        for h in range(H):
            qh = q[:, h * R:(h + 1) * R]
            kh = k[:, h * R:(h + 1) * R]
            vh = jnp.concatenate([v[:, h * R:(h + 1) * R], aug], axis=1)
            s = jax.lax.dot_general(qh, kh, (((1,), (1,)), ((), ())),
                                    preferred_element_type=jnp.float32)
            e = jnp.exp(s + bias).astype(jnp.bfloat16)
            oh = jax.lax.dot_general(e, vh, (((1,), (0,)), ((), ())),
                                     preferred_element_type=jnp.float32)
            acc_ref[:, h * AW:(h + 1) * AW] = acc_ref[:, h * AW:(h + 1) * AW] + oh

    @pl.when(j == i)
    def _finalize():
        outs = []
        for h in range(H):
            blk = acc_ref[:, h * AW:(h + 1) * AW]
            outs.append((blk[:, :R] / blk[:, R:R + 1]).astype(jnp.bfloat16))
        o = jnp.concatenate(outs, axis=1)
        out_ref[...] = jax.lax.dot_general(
            o, wo_ref[...], (((1,), (0,)), ((), ())),
            preferred_element_type=jnp.float32) + bo_ref[...]


def kernel(x, Wq, bq, Wk, bk, Wv, bv, Wo, bo):
    B = x.shape[0]
    x2 = x.reshape(S, D).astype(jnp.bfloat16)
    wq = (Wq * SCALE).astype(jnp.bfloat16)
    wk = Wk.astype(jnp.bfloat16)
    wv = Wv.astype(jnp.bfloat16)
    wo = Wo.astype(jnp.bfloat16)
    bq2 = (bq * SCALE).reshape(1, HR)
    bk2 = bk.reshape(1, HR)
    bv2 = bv.reshape(1, HR)
    bo2 = bo.reshape(1, D)

    q, k, v = pl.pallas_call(
        _qkv_kernel,
        grid=(NBQ,),
        in_specs=[
            pl.BlockSpec((BQ, D), lambda i: (i, 0)),
            pl.BlockSpec((D, HR), lambda i: (0, 0)),
            pl.BlockSpec((D, HR), lambda i: (0, 0)),
            pl.BlockSpec((D, HR), lambda i: (0, 0)),
            pl.BlockSpec((1, HR), lambda i: (0, 0)),
            pl.BlockSpec((1, HR), lambda i: (0, 0)),
            pl.BlockSpec((1, HR), lambda i: (0, 0)),
        ],
        out_specs=[
            pl.BlockSpec((BQ, HR), lambda i: (i, 0)),
            pl.BlockSpec((BQ, HR), lambda i: (i, 0)),
            pl.BlockSpec((BQ, HR), lambda i: (i, 0)),
        ],
        out_shape=[jax.ShapeDtypeStruct((S, HR), jnp.bfloat16)] * 3,
    )(x2, wq, wk, wv, bq2, bk2, bv2)

    out = pl.pallas_call(
        _attn_kernel,
        grid=(NBQ, NBK),
        in_specs=[
            pl.BlockSpec((BQ, HR), lambda i, j: (i, 0)),
            pl.BlockSpec((BK, HR), lambda i, j: (j, 0)),
            pl.BlockSpec((BK, HR), lambda i, j: (j, 0)),
            pl.BlockSpec((HR, D), lambda i, j: (0, 0)),
            pl.BlockSpec((1, D), lambda i, j: (0, 0)),
        ],
        out_specs=pl.BlockSpec((BQ, D), lambda i, j: (i, 0)),
        out_shape=jax.ShapeDtypeStruct((S, D), jnp.float32),
        scratch_shapes=[pltpu.VMEM((BQ, H * AW), jnp.float32)],
    )(q, k, v, wo, bo2)

    return out.reshape(B, S, D)
